# baseline (speedup 1.0000x reference)
import functools

import jax
import jax.numpy as jnp
from jax import lax
from jax.experimental import pallas as pl
from jax.experimental.pallas import tpu as pltpu
from jax.experimental.pallas import tpu_sc as plsc

_NC = 2
_NS = 16
_NW = _NC * _NS
_L = 16

_SENT = -1.0


def _addr_body(cbt_ref, qbt_ref, wt_ref, af_ref):
  wt = wt_ref[...]
  cf = jnp.dot(wt, cbt_ref[...].astype(jnp.float32),
               preferred_element_type=jnp.float32)
  qf = jnp.dot(wt, qbt_ref[...].astype(jnp.float32),
               preferred_element_type=jnp.float32)
  af_ref[...] = jnp.concatenate([cf, qf], axis=0).astype(jnp.int32)


def _sc_body(addr_hbm, vals_hbm, mem_hbm, out_hbm,
             abuf0, vbuf0, abuf1, vbuf1, val_loc, stage, tbuf, mem_loc,
             qbuf0, qbuf1, obuf0, obuf1,
             sem0, sem1, semq0, semq1, semo0, semo1,
             *, chunk, win, ncells, tpn, nn):
  c = lax.axis_index("c")
  s = lax.axis_index("s")
  wid = c * _NS + s
  neuron = wid // tpn
  qrow = nn + neuron
  quarter = wid % tpn
  nvec = win // _L
  nwin = chunk // win
  base = quarter * chunk

  def init_body(i, _):
    val_loc[pl.ds(i * _L, _L)] = jnp.full((_L,), _SENT, jnp.float32)
    return 0
  lax.fori_loop(0, ncells // _L, init_body, 0)

  def cfetch(w, ab, vb, sem):
    pltpu.async_copy(addr_hbm.at[neuron, pl.ds(base + w * win, win)], ab, sem)
    pltpu.async_copy(vals_hbm.at[neuron, pl.ds(base + w * win, win)], vb, sem)

  def cwait(w, ab, vb, sem):
    pltpu.make_async_copy(
        addr_hbm.at[neuron, pl.ds(base + w * win, win)], ab, sem).wait()
    pltpu.make_async_copy(
        vals_hbm.at[neuron, pl.ds(base + w * win, win)], vb, sem).wait()

  def commit_window(ab, vb):
    def vec_body(i, _):
      addrs = [ab[pl.ds((i * 4 + u) * _L, _L)] for u in range(4)]
      vals = [vb[pl.ds((i * 4 + u) * _L, _L)] for u in range(4)]
      lasts = [plsc.scan_count(a)[1] for a in addrs]
      for u in range(4):
        plsc.store_scatter(val_loc, [addrs[u]], vals[u], mask=lasts[u])
      return 0
    lax.fori_loop(0, nvec // 4, vec_body, 0)

  cfetch(0, abuf0, vbuf0, sem0)

  def cbody(g, _):
    w0 = 2 * g
    w1 = 2 * g + 1
    wn = jnp.minimum(2 * g + 2, nwin - 1)
    cwait(w0, abuf0, vbuf0, sem0)
    cfetch(w1, abuf1, vbuf1, sem1)
    commit_window(abuf0, vbuf0)
    cwait(w1, abuf1, vbuf1, sem1)
    cfetch(wn, abuf0, vbuf0, sem0)
    commit_window(abuf1, vbuf1)
    return 0
  lax.fori_loop(0, nwin // 2, cbody, 0)
  cwait(nwin - 1, abuf0, vbuf0, sem0)

  pltpu.sync_copy(val_loc, stage.at[s])
  plsc.subcore_barrier()
  pltpu.sync_copy(mem_hbm.at[neuron], mem_loc)
  grp = (s // tpn) * tpn
  for t in range(tpn):
    pltpu.sync_copy(stage.at[grp + t], tbuf)

    def mrg(i, _):
      for u in range(4):
        v = tbuf[pl.ds((i * 4 + u) * _L, _L)]
        m = mem_loc[pl.ds((i * 4 + u) * _L, _L)]
        mem_loc[pl.ds((i * 4 + u) * _L, _L)] = jnp.where(v >= 0.0, v, m)
      return 0
    lax.fori_loop(0, ncells // _L // 4, mrg, 0)

  def qfetch(w, qb, sem):
    pltpu.async_copy(addr_hbm.at[qrow, pl.ds(base + w * win, win)], qb, sem)

  def qwait(w, qb, sem):
    pltpu.make_async_copy(
        addr_hbm.at[qrow, pl.ds(base + w * win, win)], qb, sem).wait()

  def ostart(w, ob, sem):
    pltpu.async_copy(ob, out_hbm.at[neuron, pl.ds(base + w * win, win)], sem)

  def owait(w, ob, sem):
    pltpu.make_async_copy(
        ob, out_hbm.at[neuron, pl.ds(base + w * win, win)], sem).wait()

  def query_window(qb, ob):
    def vec_body(i, _):
      qs = [qb[pl.ds((i * 4 + u) * _L, _L)] for u in range(4)]
      rs = [plsc.load_gather(mem_loc, [q]) for q in qs]
      for u in range(4):
        ob[pl.ds((i * 4 + u) * _L, _L)] = rs[u]
      return 0
    lax.fori_loop(0, nvec // 4, vec_body, 0)

  qfetch(0, qbuf0, semq0)

  def qbody(g, _):
    w0 = 2 * g
    w1 = 2 * g + 1
    wn = jnp.minimum(2 * g + 2, nwin - 1)
    qwait(w0, qbuf0, semq0)
    qfetch(w1, qbuf1, semq1)

    @pl.when(g > 0)
    def _():
      owait(w0 - 2, obuf0, semo0)
    query_window(qbuf0, obuf0)
    ostart(w0, obuf0, semo0)

    qwait(w1, qbuf1, semq1)
    qfetch(wn, qbuf0, semq0)

    @pl.when(g > 0)
    def _():
      owait(w1 - 2, obuf1, semo1)
    query_window(qbuf1, obuf1)
    ostart(w1, obuf1, semo1)
    return 0
  lax.fori_loop(0, nwin // 2, qbody, 0)
  qwait(nwin - 1, qbuf0, semq0)
  owait(nwin - 2, obuf0, semo0)
  owait(nwin - 1, obuf1, semo1)


def kernel(memory, commit_bits, commit_vals, query_bits, bit_sel):
  n, ncells = memory.shape
  b, ib = commit_bits.shape
  nb = bit_sel.shape[1]
  tpn = _NW // n
  chunk = b // tpn
  win = 4096
  blk = 16384

  cbt = commit_bits.T
  qbt = query_bits.T
  valt = commit_vals.T

  pow2 = 2.0 ** jnp.arange(nb, dtype=jnp.float32)
  onehot = (bit_sel[:, :, None] ==
            jnp.arange(ib, dtype=bit_sel.dtype)[None, None, :])
  wt = jnp.einsum("nki,k->ni", onehot.astype(jnp.float32), pow2)

  addrs = pl.pallas_call(
      _addr_body,
      grid=(b // blk,),
      in_specs=[
          pl.BlockSpec((ib, blk), lambda i: (0, i)),
          pl.BlockSpec((ib, blk), lambda i: (0, i)),
          pl.BlockSpec((n, ib), lambda i: (0, 0)),
      ],
      out_specs=pl.BlockSpec((2 * n, blk), lambda i: (0, i)),
      out_shape=jax.ShapeDtypeStruct((2 * n, b), jnp.int32),
  )(cbt, qbt, wt)

  mesh = plsc.VectorSubcoreMesh(core_axis_name="c", subcore_axis_name="s",
                                num_cores=_NC, num_subcores=_NS)

  sc_call = pl.kernel(
      functools.partial(_sc_body, chunk=chunk, win=win, ncells=ncells,
                        tpn=tpn, nn=n),
      out_type=jax.ShapeDtypeStruct((n, b), jnp.float32),
      mesh=mesh,
      compiler_params=pltpu.CompilerParams(needs_layout_passes=False),
      scratch_types=[
          pltpu.VMEM((win,), jnp.int32),
          pltpu.VMEM((win,), jnp.float32),
          pltpu.VMEM((win,), jnp.int32),
          pltpu.VMEM((win,), jnp.float32),
          pltpu.VMEM((ncells,), jnp.float32),
          pltpu.VMEM_SHARED((_NS, ncells), jnp.float32),
          pltpu.VMEM((ncells,), jnp.float32),
          pltpu.VMEM((ncells,), jnp.float32),
          pltpu.VMEM((win,), jnp.int32),
          pltpu.VMEM((win,), jnp.int32),
          pltpu.VMEM((win,), jnp.float32),
          pltpu.VMEM((win,), jnp.float32),
          pltpu.SemaphoreType.DMA,
          pltpu.SemaphoreType.DMA,
          pltpu.SemaphoreType.DMA,
          pltpu.SemaphoreType.DMA,
          pltpu.SemaphoreType.DMA,
          pltpu.SemaphoreType.DMA,
      ],
  )
  out = sc_call(addrs, valt, memory)
  return out.T


# speedup vs baseline: 1.0573x; 1.0573x over previous
import functools

import jax
import jax.numpy as jnp
from jax import lax
from jax.experimental import pallas as pl
from jax.experimental.pallas import tpu as pltpu
from jax.experimental.pallas import tpu_sc as plsc

_NC = 2
_NS = 16
_NW = _NC * _NS
_L = 16

_SENT = -1.0


def _addr_body(bt_ref, wt_ref, af_ref):
  wt = wt_ref[...]
  af = jnp.dot(wt, bt_ref[...].astype(jnp.float32),
               preferred_element_type=jnp.float32)
  af_ref[...] = af.astype(jnp.int32)


def _commit_body(addr_hbm, vals_hbm, part_hbm,
                 abuf0, vbuf0, abuf1, vbuf1, val_loc, stage, tbuf,
                 sem0, sem1,
                 *, chunk, win, ncells, tpn):
  c = lax.axis_index("c")
  s = lax.axis_index("s")
  wid = c * _NS + s
  neuron = wid // tpn
  quarter = wid % tpn
  nvec = win // _L
  nwin = chunk // win
  base = quarter * chunk

  def init_body(i, _):
    val_loc[pl.ds(i * _L, _L)] = jnp.full((_L,), _SENT, jnp.float32)
    return 0
  lax.fori_loop(0, ncells // _L, init_body, 0)

  def cfetch(w, ab, vb, sem):
    pltpu.async_copy(addr_hbm.at[neuron, pl.ds(base + w * win, win)], ab, sem)
    pltpu.async_copy(vals_hbm.at[neuron, pl.ds(base + w * win, win)], vb, sem)

  def cwait(w, ab, vb, sem):
    pltpu.make_async_copy(
        addr_hbm.at[neuron, pl.ds(base + w * win, win)], ab, sem).wait()
    pltpu.make_async_copy(
        vals_hbm.at[neuron, pl.ds(base + w * win, win)], vb, sem).wait()

  def commit_window(ab, vb):
    def vec_body(i, _):
      addrs = [ab[pl.ds((i * 4 + u) * _L, _L)] for u in range(4)]
      vals = [vb[pl.ds((i * 4 + u) * _L, _L)] for u in range(4)]
      lasts = [plsc.scan_count(a)[1] for a in addrs]
      for u in range(4):
        plsc.store_scatter(val_loc, [addrs[u]], vals[u], mask=lasts[u])
      return 0
    lax.fori_loop(0, nvec // 4, vec_body, 0)

  cfetch(0, abuf0, vbuf0, sem0)

  def cbody(g, _):
    w0 = 2 * g
    w1 = 2 * g + 1
    wn = jnp.minimum(2 * g + 2, nwin - 1)
    cwait(w0, abuf0, vbuf0, sem0)
    cfetch(w1, abuf1, vbuf1, sem1)
    commit_window(abuf0, vbuf0)
    cwait(w1, abuf1, vbuf1, sem1)
    cfetch(wn, abuf0, vbuf0, sem0)
    commit_window(abuf1, vbuf1)
    return 0
  lax.fori_loop(0, nwin // 2, cbody, 0)
  cwait(nwin - 1, abuf0, vbuf0, sem0)

  pltpu.sync_copy(val_loc, stage.at[s])
  plsc.subcore_barrier()
  grp = (s // tpn) * tpn
  nloc = s // tpn
  piece = s % tpn
  psize = ncells // tpn
  pbase = piece * psize
  pltpu.sync_copy(stage.at[grp, pl.ds(pbase, psize)], tbuf.at[pl.ds(0, psize)])
  for t in range(1, tpn):
    pltpu.sync_copy(stage.at[grp + t, pl.ds(pbase, psize)],
                    val_loc.at[pl.ds(0, psize)])

    def mrg(i, _):
      for u in range(4):
        v = val_loc[pl.ds((i * 4 + u) * _L, _L)]
        m = tbuf[pl.ds((i * 4 + u) * _L, _L)]
        tbuf[pl.ds((i * 4 + u) * _L, _L)] = jnp.where(v >= 0.0, v, m)
      return 0
    lax.fori_loop(0, psize // _L // 4, mrg, 0)
  pltpu.sync_copy(tbuf.at[pl.ds(0, psize)],
                  part_hbm.at[neuron, pl.ds(pbase, psize)])


def _query_body(part_hbm, mem_hbm, qaddr_hbm, out_hbm,
                tbuf, mem_loc, qbuf0, qbuf1, obuf0, obuf1,
                semq0, semq1, semo0, semo1,
                *, chunk, win, ncells, tpn):
  c = lax.axis_index("c")
  s = lax.axis_index("s")
  wid = c * _NS + s
  neuron = wid // tpn
  quarter = wid % tpn
  nvec = win // _L
  nwin = chunk // win
  base = quarter * chunk

  pltpu.sync_copy(part_hbm.at[neuron], tbuf)
  pltpu.sync_copy(mem_hbm.at[neuron], mem_loc)

  def mrg(i, _):
    for u in range(4):
      v = tbuf[pl.ds((i * 4 + u) * _L, _L)]
      m = mem_loc[pl.ds((i * 4 + u) * _L, _L)]
      mem_loc[pl.ds((i * 4 + u) * _L, _L)] = jnp.where(v >= 0.0, v, m)
    return 0
  lax.fori_loop(0, ncells // _L // 4, mrg, 0)

  def qfetch(w, qb, sem):
    pltpu.async_copy(qaddr_hbm.at[neuron, pl.ds(base + w * win, win)], qb, sem)

  def qwait(w, qb, sem):
    pltpu.make_async_copy(
        qaddr_hbm.at[neuron, pl.ds(base + w * win, win)], qb, sem).wait()

  def ostart(w, ob, sem):
    pltpu.async_copy(ob, out_hbm.at[neuron, pl.ds(base + w * win, win)], sem)

  def owait(w, ob, sem):
    pltpu.make_async_copy(
        ob, out_hbm.at[neuron, pl.ds(base + w * win, win)], sem).wait()

  def query_window(qb, ob):
    def vec_body(i, _):
      qs = [qb[pl.ds((i * 4 + u) * _L, _L)] for u in range(4)]
      rs = [plsc.load_gather(mem_loc, [q]) for q in qs]
      for u in range(4):
        ob[pl.ds((i * 4 + u) * _L, _L)] = rs[u]
      return 0
    lax.fori_loop(0, nvec // 4, vec_body, 0)

  qfetch(0, qbuf0, semq0)

  def qbody(g, _):
    w0 = 2 * g
    w1 = 2 * g + 1
    wn = jnp.minimum(2 * g + 2, nwin - 1)
    qwait(w0, qbuf0, semq0)
    qfetch(w1, qbuf1, semq1)

    @pl.when(g > 0)
    def _():
      owait(w0 - 2, obuf0, semo0)
    query_window(qbuf0, obuf0)
    ostart(w0, obuf0, semo0)

    qwait(w1, qbuf1, semq1)
    qfetch(wn, qbuf0, semq0)

    @pl.when(g > 0)
    def _():
      owait(w1 - 2, obuf1, semo1)
    query_window(qbuf1, obuf1)
    ostart(w1, obuf1, semo1)
    return 0
  lax.fori_loop(0, nwin // 2, qbody, 0)
  qwait(nwin - 1, qbuf0, semq0)
  owait(nwin - 2, obuf0, semo0)
  owait(nwin - 1, obuf1, semo1)


def kernel(memory, commit_bits, commit_vals, query_bits, bit_sel):
  n, ncells = memory.shape
  b, ib = commit_bits.shape
  nb = bit_sel.shape[1]
  tpn = _NW // n
  chunk = b // tpn
  win = 4096
  blk = 16384

  cbt = commit_bits.T
  qbt = query_bits.T
  valt = commit_vals.T

  pow2 = 2.0 ** jnp.arange(nb, dtype=jnp.float32)
  onehot = (bit_sel[:, :, None] ==
            jnp.arange(ib, dtype=bit_sel.dtype)[None, None, :])
  wt = jnp.einsum("nki,k->ni", onehot.astype(jnp.float32), pow2)

  addr_call = pl.pallas_call(
      _addr_body,
      grid=(b // blk,),
      in_specs=[
          pl.BlockSpec((ib, blk), lambda i: (0, i)),
          pl.BlockSpec((n, ib), lambda i: (0, 0)),
      ],
      out_specs=pl.BlockSpec((n, blk), lambda i: (0, i)),
      out_shape=jax.ShapeDtypeStruct((n, b), jnp.int32),
  )
  caddr = addr_call(cbt, wt)
  qaddr = addr_call(qbt, wt)

  mesh = plsc.VectorSubcoreMesh(core_axis_name="c", subcore_axis_name="s",
                                num_cores=_NC, num_subcores=_NS)

  commit_call = pl.kernel(
      functools.partial(_commit_body, chunk=chunk, win=win, ncells=ncells,
                        tpn=tpn),
      out_type=jax.ShapeDtypeStruct((n, ncells), jnp.float32),
      mesh=mesh,
      compiler_params=pltpu.CompilerParams(needs_layout_passes=False),
      scratch_types=[
          pltpu.VMEM((win,), jnp.int32),
          pltpu.VMEM((win,), jnp.float32),
          pltpu.VMEM((win,), jnp.int32),
          pltpu.VMEM((win,), jnp.float32),
          pltpu.VMEM((ncells,), jnp.float32),
          pltpu.VMEM_SHARED((_NS, ncells), jnp.float32),
          pltpu.VMEM((ncells,), jnp.float32),
          pltpu.SemaphoreType.DMA,
          pltpu.SemaphoreType.DMA,
      ],
  )
  part = commit_call(caddr, valt)

  query_call = pl.kernel(
      functools.partial(_query_body, chunk=chunk, win=win, ncells=ncells,
                        tpn=tpn),
      out_type=jax.ShapeDtypeStruct((n, b), jnp.float32),
      mesh=mesh,
      compiler_params=pltpu.CompilerParams(needs_layout_passes=False),
      scratch_types=[
          pltpu.VMEM((ncells,), jnp.float32),
          pltpu.VMEM((ncells,), jnp.float32),
          pltpu.VMEM((win,), jnp.int32),
          pltpu.VMEM((win,), jnp.int32),
          pltpu.VMEM((win,), jnp.float32),
          pltpu.VMEM((win,), jnp.float32),
          pltpu.SemaphoreType.DMA,
          pltpu.SemaphoreType.DMA,
          pltpu.SemaphoreType.DMA,
          pltpu.SemaphoreType.DMA,
      ],
  )
  out = query_call(part, memory, qaddr)
  return out.T
